# baseline (speedup 1.0000x reference)
import functools

import jax
import jax.numpy as jnp
from jax import lax
from jax.experimental import pallas as pl
from jax.experimental.pallas import tpu as pltpu
from jax.experimental.pallas import tpu_sc as plsc

LANES = 16
NW = 32


def _tc_pack_table(V3):
    D = V3.shape[0]
    H = V3.shape[2]
    RB = 2048
    grid = (H + RB - 1) // RB

    def body(a_ref, o_ref):
        o_ref[:, 0:D] = a_ref[:, 0, :].T
        o_ref[:, D:2 * D] = a_ref[:, 1, :].T

    return pl.pallas_call(
        body,
        grid=(grid,),
        in_specs=[pl.BlockSpec((D, 2, RB), lambda i: (0, 0, i))],
        out_specs=pl.BlockSpec((RB, 2 * D), lambda i: (i, 0)),
        out_shape=jax.ShapeDtypeStruct((H, 2 * D), jnp.float32),
    )(V3)


def _sc_scores(T, u_all, idx_all, n_pairs, bpw):
    D = 64
    K = n_pairs - 1
    G = bpw // LANES
    neg_per_group = K * LANES
    rows_per_group = n_pairs * LANES
    urows_g = LANES * D // 128
    orows_w = n_pairs * bpw // 128

    mesh = plsc.VectorSubcoreMesh(core_axis_name="c", subcore_axis_name="s")

    @functools.partial(
        pl.kernel,
        mesh=mesh,
        compiler_params=pltpu.CompilerParams(
            needs_layout_passes=False, use_tc_tiling_on_sc=False
        ),
        out_type=jax.ShapeDtypeStruct((NW * orows_w, 128), jnp.float32),
        scratch_types=[
            pltpu.VMEM((G * 6, 128), jnp.int32),
            pltpu.VMEM((urows_g, 128), jnp.float32),
            pltpu.VMEM((rows_per_group, 128), jnp.float32),
            pltpu.VMEM((orows_w, 128), jnp.float32),
            pltpu.SemaphoreType.DMA,
        ],
    )
    def k(T_hbm, u_hbm, idx_hbm, out_hbm, idx_v, u_v, rows_v, sc_v, sem):
        wid = lax.axis_index("s") * 2 + lax.axis_index("c")
        pltpu.sync_copy(idx_hbm.at[pl.ds(wid * (G * 6), G * 6), :], idx_v)
        iota = lax.iota(jnp.int32, LANES)
        iota_k = iota * K
        urow = iota // 2
        ucolb = (iota & 1) * D

        def issue(g, sem):
            pltpu.async_copy(
                T_hbm.at[idx_v.at[6 * g]], rows_v.at[pl.ds(0, 128)], sem
            )
            pltpu.async_copy(
                T_hbm.at[idx_v.at[6 * g + 1]], rows_v.at[pl.ds(128, 128)], sem
            )
            pltpu.async_copy(
                T_hbm.at[idx_v.at[6 * g + 2, pl.ds(0, 80)]],
                rows_v.at[pl.ds(256, 80)],
                sem,
            )
            pltpu.async_copy(
                u_hbm.at[pl.ds(wid * (G * urows_g) + g * urows_g, urows_g), :],
                u_v,
                sem,
            )

        def drain(sem):
            for rng in ((0, 128), (128, 128), (256, 80)):
                pltpu.make_async_copy(
                    T_hbm.at[pl.ds(0, rng[1])],
                    rows_v.at[pl.ds(rng[0], rng[1])],
                    sem,
                ).wait()
            pltpu.make_async_copy(
                u_hbm.at[pl.ds(0, urows_g), :], u_v, sem
            ).wait()

        def compute(g):
            pslot = 320 + iota
            par_pos = plsc.load_gather(
                idx_v, [(6 * g + 3) + (pslot >> 7), pslot & 127]
            )
            par_negs = []
            for j in range(1, n_pairs):
                nslot = iota_k + (j - 1)
                par_negs.append(
                    plsc.load_gather(
                        idx_v, [(6 * g + 3) + (nslot >> 7), nslot & 127]
                    )
                )

            def d_body(d, accs):
                dcol = (iota + d) & (D - 1)
                ud = plsc.load_gather(u_v, [urow, ucolb + dcol])
                pos_r = plsc.load_gather(
                    rows_v, [320 + iota, par_pos + dcol]
                )
                neg_rs = [
                    plsc.load_gather(
                        rows_v, [iota_k + (j - 1), par_negs[j - 1] + dcol]
                    )
                    for j in range(1, n_pairs)
                ]
                return (accs[0] + ud * pos_r,) + tuple(
                    accs[j] + ud * neg_rs[j - 1] for j in range(1, n_pairs)
                )

            zero = jnp.zeros((LANES,), jnp.float32)
            accs = lax.fori_loop(
                0, D, d_body, tuple(zero for _ in range(n_pairs))
            )
            orow = g // 8
            ocol = (g & 7) * LANES
            sc_v[orow, pl.ds(ocol, LANES)] = accs[0]
            for j in range(1, n_pairs):
                sc_v[4 * j + orow, pl.ds(ocol, LANES)] = -accs[j]

        def body(g, carry):
            issue(g, sem)
            drain(sem)
            compute(g)
            return carry

        lax.fori_loop(0, G, body, 0)
        pltpu.sync_copy(sc_v, out_hbm.at[pl.ds(wid * orows_w, orows_w), :])

    return k(T, u_all, idx_all)


def _tc_logsig_sum(scores2d):
    def body(x_ref, o_ref):
        x = x_ref[...]
        ls = jnp.minimum(x, 0.0) - jnp.log(1.0 + jnp.exp(-jnp.abs(x)))
        o_ref[...] = (-jnp.sum(ls)).reshape(1, 1)

    out = pl.pallas_call(
        body,
        out_shape=jax.ShapeDtypeStruct((1, 1), jnp.float32),
    )(scores2d)
    return out[0, 0]


def kernel(pos_u, pos_v, neg_v, V):
    B, D = pos_u.shape
    K = neg_v.shape[1]
    n_pairs = K + 1
    bpw = B // NW
    G = bpw // LANES
    VOC = V.shape[0]
    H = VOC // 2

    pos_i = pos_v.astype(jnp.int32)
    neg_i = neg_v.astype(jnp.int32)
    pos_h = jnp.where(pos_i >= H, pos_i - H, pos_i)
    neg_h = jnp.where(neg_i >= H, neg_i - H, neg_i)
    pos_p = jnp.where(pos_i >= H, D, 0)
    neg_p = jnp.where(neg_i >= H, D, 0)

    def blocks(neg, pos):
        return jnp.concatenate(
            [
                neg.reshape(NW, G, LANES * K),
                pos.reshape(NW, G, LANES),
                jnp.zeros((NW, G, 48), jnp.int32),
            ],
            axis=-1,
        ).reshape(NW, G, 3, 128)

    idx_all = jnp.concatenate(
        [blocks(neg_h, pos_h), blocks(neg_p, pos_p)], axis=2
    ).reshape(NW * G * 6, 128)
    u_all = pos_u.reshape(B * D // 128, 128)

    T = _tc_pack_table(V.T.reshape(D, 2, H))
    scores = _sc_scores(T, u_all, idx_all, n_pairs, bpw)
    return _tc_logsig_sum(scores)
